# Initial kernel scaffold; baseline (speedup 1.0000x reference)
import jax
import jax.numpy as jnp
from jax.experimental import pallas as pl


def kernel(xyz1, xyz2, points1, points2, W0, b0, g0, be0, W1, b1, g1, be1):
    raise NotImplementedError("write your pallas kernel here")


# speedup vs baseline: 25.6259x; 25.6259x over previous
import jax
import jax.numpy as jnp
from jax import lax
from jax.experimental import pallas as pl
from jax.experimental.pallas import tpu as pltpu

_TILE = 512


def _stage_a(xyz1_ref, xyz2_ref, p2p_ref, p1_ref, w0t_ref, b0_ref,
             y0_ref, s0_ref, q0_ref):
    S = xyz2_ref.shape[2]
    x = xyz1_ref[0]
    yt = xyz2_ref[0]
    xx = jnp.sum(x * x, axis=1, keepdims=True)
    yy = jnp.sum(yt * yt, axis=0, keepdims=True)
    d = xx - 2.0 * jnp.dot(x, yt, preferred_element_type=jnp.float32) + yy

    ii = lax.broadcasted_iota(jnp.int32, d.shape, 1)
    BIG = jnp.float32(3.0e38)
    m1 = jnp.min(d, axis=1, keepdims=True)
    i1 = jnp.min(jnp.where(d == m1, ii, S), axis=1, keepdims=True)
    d2 = jnp.where(ii == i1, BIG, d)
    m2 = jnp.min(d2, axis=1, keepdims=True)
    i2 = jnp.min(jnp.where(d2 == m2, ii, S), axis=1, keepdims=True)
    d3 = jnp.where(ii == i2, BIG, d2)
    m3 = jnp.min(d3, axis=1, keepdims=True)
    i3 = jnp.min(jnp.where(d3 == m3, ii, S), axis=1, keepdims=True)

    r1 = 1.0 / (m1 + 1e-8)
    r2 = 1.0 / (m2 + 1e-8)
    r3 = 1.0 / (m3 + 1e-8)
    inv = 1.0 / (r1 + r2 + r3)
    zero = jnp.float32(0.0)
    wd = (jnp.where(ii == i1, r1 * inv, zero)
          + jnp.where(ii == i2, r2 * inv, zero)
          + jnp.where(ii == i3, r3 * inv, zero))

    interp = jnp.dot(wd, p2p_ref[0], preferred_element_type=jnp.float32)
    cat = jnp.concatenate([p1_ref[0], interp], axis=1)
    y0 = jnp.dot(cat, w0t_ref[...], preferred_element_type=jnp.float32) + b0_ref[...]
    y0_ref[0] = y0

    @pl.when((pl.program_id(0) == 0) & (pl.program_id(1) == 0))
    def _():
        s0_ref[...] = jnp.zeros_like(s0_ref)
        q0_ref[...] = jnp.zeros_like(q0_ref)

    s0_ref[...] += jnp.sum(y0, axis=0, keepdims=True)
    q0_ref[...] += jnp.sum(y0 * y0, axis=0, keepdims=True)


def _stage_b(y0_ref, s0_ref, q0_ref, g0_ref, be0_ref, w1t_ref, b1_ref,
             y1_ref, s1_ref, q1_ref, *, inv_m):
    mean = s0_ref[...] * inv_m
    var = q0_ref[...] * inv_m - mean * mean
    scale = g0_ref[...] * lax.rsqrt(var + 1e-5)
    shift = be0_ref[...] - mean * scale
    x1 = jnp.maximum(y0_ref[0] * scale + shift, 0.0)
    y1 = jnp.dot(x1, w1t_ref[...], preferred_element_type=jnp.float32) + b1_ref[...]
    y1_ref[0] = y1

    @pl.when((pl.program_id(0) == 0) & (pl.program_id(1) == 0))
    def _():
        s1_ref[...] = jnp.zeros_like(s1_ref)
        q1_ref[...] = jnp.zeros_like(q1_ref)

    s1_ref[...] += jnp.sum(y1, axis=0, keepdims=True)
    q1_ref[...] += jnp.sum(y1 * y1, axis=0, keepdims=True)


def _stage_c(y1_ref, s1_ref, q1_ref, g1_ref, be1_ref, out_ref, *, inv_m):
    mean = s1_ref[...] * inv_m
    var = q1_ref[...] * inv_m - mean * mean
    scale = g1_ref[...] * lax.rsqrt(var + 1e-5)
    shift = be1_ref[...] - mean * scale
    x2 = jnp.maximum(y1_ref[0] * scale + shift, 0.0)
    out_ref[0] = x2.T


def kernel(xyz1, xyz2, points1, points2, W0, b0, g0, be0, W1, b1, g1, be1):
    import functools

    B, N, _ = xyz1.shape
    S = xyz2.shape[2]
    D1 = points1.shape[2]
    D2 = points2.shape[1]
    C0 = W0.shape[0]
    C1 = W1.shape[0]
    T = _TILE
    NT = N // T
    inv_m = 1.0 / float(B * N)

    p2p = jnp.transpose(points2, (0, 2, 1))
    w0t = W0.T
    w1t = W1.T
    b0r, g0r, be0r = b0.reshape(1, C0), g0.reshape(1, C0), be0.reshape(1, C0)
    b1r, g1r, be1r = b1.reshape(1, C1), g1.reshape(1, C1), be1.reshape(1, C1)

    stats_spec_c0 = pl.BlockSpec((1, C0), lambda b, n: (0, 0))
    stats_spec_c1 = pl.BlockSpec((1, C1), lambda b, n: (0, 0))
    params = pltpu.CompilerParams(dimension_semantics=("arbitrary", "arbitrary"))

    y0, s0, q0 = pl.pallas_call(
        _stage_a,
        grid=(B, NT),
        in_specs=[
            pl.BlockSpec((1, T, 3), lambda b, n: (b, n, 0)),
            pl.BlockSpec((1, 3, S), lambda b, n: (b, 0, 0)),
            pl.BlockSpec((1, S, D2), lambda b, n: (b, 0, 0)),
            pl.BlockSpec((1, T, D1), lambda b, n: (b, n, 0)),
            pl.BlockSpec((D1 + D2, C0), lambda b, n: (0, 0)),
            stats_spec_c0,
        ],
        out_specs=[
            pl.BlockSpec((1, T, C0), lambda b, n: (b, n, 0)),
            stats_spec_c0,
            stats_spec_c0,
        ],
        out_shape=[
            jax.ShapeDtypeStruct((B, N, C0), jnp.float32),
            jax.ShapeDtypeStruct((1, C0), jnp.float32),
            jax.ShapeDtypeStruct((1, C0), jnp.float32),
        ],
        compiler_params=params,
    )(xyz1, xyz2, p2p, points1, w0t, b0r)

    y1, s1, q1 = pl.pallas_call(
        functools.partial(_stage_b, inv_m=inv_m),
        grid=(B, NT),
        in_specs=[
            pl.BlockSpec((1, T, C0), lambda b, n: (b, n, 0)),
            stats_spec_c0,
            stats_spec_c0,
            stats_spec_c0,
            stats_spec_c0,
            pl.BlockSpec((C0, C1), lambda b, n: (0, 0)),
            stats_spec_c1,
        ],
        out_specs=[
            pl.BlockSpec((1, T, C1), lambda b, n: (b, n, 0)),
            stats_spec_c1,
            stats_spec_c1,
        ],
        out_shape=[
            jax.ShapeDtypeStruct((B, N, C1), jnp.float32),
            jax.ShapeDtypeStruct((1, C1), jnp.float32),
            jax.ShapeDtypeStruct((1, C1), jnp.float32),
        ],
        compiler_params=params,
    )(y0, s0, q0, g0r, be0r, w1t, b1r)

    out = pl.pallas_call(
        functools.partial(_stage_c, inv_m=inv_m),
        grid=(B, NT),
        in_specs=[
            pl.BlockSpec((1, T, C1), lambda b, n: (b, n, 0)),
            stats_spec_c1,
            stats_spec_c1,
            stats_spec_c1,
            stats_spec_c1,
        ],
        out_specs=pl.BlockSpec((1, C1, T), lambda b, n: (b, 0, n)),
        out_shape=jax.ShapeDtypeStruct((B, C1, N), jnp.float32),
        compiler_params=params,
    )(y1, s1, q1, g1r, be1r)

    return out
